# Initial kernel scaffold; baseline (speedup 1.0000x reference)
import jax
import jax.numpy as jnp
from jax.experimental import pallas as pl


def kernel(x, target):
    raise NotImplementedError("write your pallas kernel here")


# speedup vs baseline: 8.5174x; 8.5174x over previous
import functools
import math

import jax
import jax.numpy as jnp
from jax.experimental import pallas as pl
from jax.experimental.pallas import tpu as pltpu

_SIZE = 32000
_PAD = 0
_SMOOTH = 0.1
_CONF = 1.0 - _SMOOTH
_EPS = _SMOOTH / (_SIZE - 2)
_C = _CONF * math.log(_CONF) + _SMOOTH * math.log(_EPS)

_ROWS_BLK = 128


def _loss_body(t_ref, x_ref, out_ref):
    i = pl.program_id(0)

    @pl.when(i == 0)
    def _init():
        out_ref[0, 0] = 0.0

    x = x_ref[...]
    t = t_ref[...]
    col = jax.lax.broadcasted_iota(jnp.int32, x.shape, 1)
    w = jnp.where(col == t, _CONF, _EPS)
    wsum = jnp.sum(x * w, axis=1)
    z = x[:, 0]
    m = (t[:, 0] != _PAD).astype(jnp.float32)
    contrib = m * (_C - wsum + _EPS * z)
    out_ref[0, 0] += jnp.sum(contrib)


@jax.jit
def kernel(x, target):
    n = x.shape[0]
    t2 = target.astype(jnp.int32).reshape(n, 1)
    out = pl.pallas_call(
        _loss_body,
        grid=(n // _ROWS_BLK,),
        in_specs=[
            pl.BlockSpec((_ROWS_BLK, 1), lambda i: (i, 0)),
            pl.BlockSpec((_ROWS_BLK, _SIZE), lambda i: (i, 0)),
        ],
        out_specs=pl.BlockSpec(
            (1, 1), lambda i: (0, 0), memory_space=pltpu.SMEM
        ),
        out_shape=jax.ShapeDtypeStruct((1, 1), jnp.float32),
    )(t2, x)
    return out[0, 0]
